# Initial kernel scaffold; baseline (speedup 1.0000x reference)
import jax
import jax.numpy as jnp
from jax.experimental import pallas as pl


def kernel(edge_attr, edge_index, batch, W_rel1, b_rel1, W_root1, W_rel2, b_rel2, W_root2, W_rel3, b_rel3, W_root3, W_rel4, b_rel4, W_root4, W5, b5, W6, b6, W7, b7, W8, b8, W_out, b_out):
    raise NotImplementedError("write your pallas kernel here")


# speedup vs baseline: 2.5488x; 2.5488x over previous
import dataclasses

import jax
import jax.numpy as jnp
from jax import lax
from jax.experimental import pallas as pl
from jax.experimental.pallas import tpu as pltpu
from jax.experimental.pallas import tpu_sc as plsc

N = 10000
E = 160000
HID = 256
HALF = 128
G = 64
NPAD = 10240
ROWS = NPAD // 16
EPT = NPAD
EPAD = 16 * EPT
DTRASH = 10200
EBS = 128
NBS = EPT // EBS
EBP = 512
NBP = EPT // EBP
GPAD = 72


def _sc_seg_body(x_hbm, srcp_hbm, dstp_hbm, zeros_hbm, agg_out,
                 acc, isrc, idst, rows):
    c = lax.axis_index("c")
    s = lax.axis_index("s")
    rowoff = c * NPAD
    pltpu.sync_copy(zeros_hbm, acc.at[pl.ds(s * ROWS, ROWS)])
    plsc.subcore_barrier()

    @pl.loop(0, NBS)
    def _(i):
        base = s * EPT + i * EBS
        pltpu.sync_copy(srcp_hbm.at[pl.ds(base, EBS)], isrc)
        pltpu.sync_copy(dstp_hbm.at[pl.ds(base, EBS)], idst)

        @pl.loop(0, EBS // 16)
        def _(j):
            isrc[pl.ds(j * 16, 16)] = isrc[pl.ds(j * 16, 16)] + rowoff

        pltpu.sync_copy(x_hbm.at[isrc], rows)
        pltpu.sync_copy(rows, acc.at[idst], add=True)

    plsc.subcore_barrier()
    pltpu.sync_copy(acc.at[pl.ds(s * ROWS, ROWS)],
                    agg_out.at[pl.ds(rowoff + s * ROWS, ROWS)])


def _sc_pool_body(x_hbm, srcp_hbm, dstp_hbm, batchpad_hbm, ones_hbm, zeros_hbm,
                  pool_out,
                  accP, accX, accC, bpad_v, isrc, idst, bgbuf, bgn_v, rows):
    c = lax.axis_index("c")
    s = lax.axis_index("s")
    rowoff = c * NPAD

    @pl.when(s == 0)
    def _():
        pltpu.sync_copy(zeros_hbm, accP)

    @pl.when(s == 1)
    def _():
        pltpu.sync_copy(zeros_hbm, accX)

    @pl.when(s == 2)
    def _():
        pltpu.sync_copy(zeros_hbm, accC)

    pltpu.sync_copy(batchpad_hbm, bpad_v)
    plsc.subcore_barrier()

    @pl.loop(0, NBP)
    def _(i):
        base = s * EPT + i * EBP
        pltpu.sync_copy(srcp_hbm.at[pl.ds(base, EBP)], isrc)
        pltpu.sync_copy(dstp_hbm.at[pl.ds(base, EBP)], idst)

        @pl.loop(0, EBP // 16)
        def _(j):
            d16 = idst[pl.ds(j * 16, 16)]
            bg16 = plsc.load_gather(bpad_v, [d16])
            bgbuf[pl.ds(j * 16, 16)] = bg16
            isrc[pl.ds(j * 16, 16)] = isrc[pl.ds(j * 16, 16)] + rowoff

        pltpu.sync_copy(x_hbm.at[isrc], rows.at[pl.ds(0, EBP)])
        pltpu.sync_copy(rows.at[pl.ds(0, EBP)], accP.at[bgbuf], add=True)

    pltpu.sync_copy(batchpad_hbm.at[pl.ds(s * ROWS, ROWS)], bgn_v)
    pltpu.sync_copy(x_hbm.at[pl.ds(rowoff + s * ROWS, ROWS)], rows.at[pl.ds(0, ROWS)])
    pltpu.sync_copy(rows.at[pl.ds(0, ROWS)], accX.at[bgn_v], add=True)

    @pl.when(c == 0)
    def _():
        pltpu.sync_copy(ones_hbm, rows.at[pl.ds(0, ROWS)])
        pltpu.sync_copy(rows.at[pl.ds(0, ROWS)], accC.at[bgn_v], add=True)

    plsc.subcore_barrier()

    @pl.when(s == 0)
    def _():
        pltpu.sync_copy(accP.at[pl.ds(0, G)], pool_out.at[pl.ds(c * 192, G)])

    @pl.when(s == 1)
    def _():
        pltpu.sync_copy(accX.at[pl.ds(0, G)], pool_out.at[pl.ds(c * 192 + G, G)])

    @pl.when(s == 2)
    def _():
        pltpu.sync_copy(accC.at[pl.ds(0, G)], pool_out.at[pl.ds(c * 192 + 2 * G, G)])


def _tc_bcast_body(ea_ref, out_ref):
    out_ref[...] = jnp.broadcast_to(ea_ref[...], out_ref.shape)


def _tc1_body(agg1_ref, ea_ref, wrel_ref, wroot_ref, b_ref, out_ref):
    a = agg1_ref[...][:, :1]
    e = ea_ref[...]
    out_ref[...] = jnp.maximum(a * wrel_ref[...] + e * wroot_ref[...] + b_ref[...], 0.0)


def _tc_layer_body(agglo, agghi, xlo, xhi, wrel, wroot, b, out_ref):
    agg = jnp.concatenate([agglo[...], agghi[...]], axis=1)
    x = jnp.concatenate([xlo[...], xhi[...]], axis=1)
    res = (jnp.dot(agg, wrel[...], preferred_element_type=jnp.float32,
                   precision=lax.Precision.HIGHEST)
           + jnp.dot(x, wroot[...], preferred_element_type=jnp.float32,
                     precision=lax.Precision.HIGHEST)
           + b[...])
    out_ref[...] = jnp.maximum(res, 0.0)


def _tc_head_body(pool_ref, wrel4, wroot4, brel4,
                  w5, b5, w6, b6, w7, b7, w8, b8, wout, bout, out_ref):
    pool = pool_ref[...]
    P = jnp.concatenate([pool[0:G], pool[192:192 + G]], axis=1)
    X = jnp.concatenate([pool[G:2 * G], pool[192 + G:192 + 2 * G]], axis=1)
    cnt = pool[2 * G:3 * G, :1]

    def dot(a, b):
        return jnp.dot(a, b, preferred_element_type=jnp.float32,
                       precision=lax.Precision.HIGHEST)

    sums = dot(P, wrel4[...]) + cnt * brel4[...] + dot(X, wroot4[...])
    g = sums / jnp.maximum(cnt, 1.0)
    g = dot(g, w5[...]) + b5[...]
    g = dot(g, w6[...]) + b6[...]
    g = dot(g, w7[...]) + b7[...]
    g = dot(g, w8[...]) + b8[...]
    out_ref[...] = jax.nn.sigmoid(dot(g, wout[...]) + bout[...])


def _full(shape):
    return pl.BlockSpec(shape, lambda *_: tuple(0 for _ in shape))


def kernel(edge_attr, edge_index, batch,
           W_rel1, b_rel1, W_root1,
           W_rel2, b_rel2, W_root2,
           W_rel3, b_rel3, W_root3,
           W_rel4, b_rel4, W_root4,
           W5, b5, W6, b6, W7, b7, W8, b8,
           W_out, b_out):
    srcp = jnp.concatenate([edge_index[0], jnp.zeros((EPAD - E,), jnp.int32)])
    dstp = jnp.concatenate([edge_index[1], jnp.full((EPAD - E,), DTRASH, jnp.int32)])
    batchpad = jnp.concatenate([batch, jnp.full((NPAD - N,), G, jnp.int32)])
    eap = jnp.pad(edge_attr, ((0, NPAD - N), (0, 0)))
    ones_rows = jnp.ones((ROWS, HALF), jnp.float32)
    zeros_rows = jnp.zeros((ROWS, HALF), jnp.float32)
    zeros_g = jnp.zeros((GPAD, HALF), jnp.float32)

    mesh = plsc.VectorSubcoreMesh(core_axis_name="c", subcore_axis_name="s")
    sc_params = pltpu.CompilerParams()
    if "needs_layout_passes" in pltpu.CompilerParams.__dataclass_fields__:
        sc_params = dataclasses.replace(sc_params, needs_layout_passes=False)

    sc_seg = pl.kernel(
        _sc_seg_body,
        out_type=jax.ShapeDtypeStruct((2 * NPAD, HALF), jnp.float32),
        mesh=mesh,
        scratch_types=[pltpu.VMEM_SHARED((NPAD, HALF), jnp.float32),
                       pltpu.VMEM((EBS,), jnp.int32),
                       pltpu.VMEM((EBS,), jnp.int32),
                       pltpu.VMEM((EBS, HALF), jnp.float32)],
        compiler_params=sc_params,
    )

    tc_bcast = pl.pallas_call(
        _tc_bcast_body,
        grid=(10, 2),
        in_specs=[pl.BlockSpec((1024, 1), lambda i, h: (i, 0))],
        out_specs=pl.BlockSpec((1024, HALF), lambda i, h: (h * 10 + i, 0)),
        out_shape=jax.ShapeDtypeStruct((2 * NPAD, HALF), jnp.float32),
    )
    ea128 = tc_bcast(eap)
    agg1_st = sc_seg(ea128, srcp, dstp, zeros_rows)

    tc1 = pl.pallas_call(
        _tc1_body,
        grid=(10, 2),
        in_specs=[pl.BlockSpec((1024, HALF), lambda i, h: (i, 0)),
                  pl.BlockSpec((1024, 1), lambda i, h: (i, 0)),
                  pl.BlockSpec((1, HALF), lambda i, h: (0, h)),
                  pl.BlockSpec((1, HALF), lambda i, h: (0, h)),
                  pl.BlockSpec((1, HALF), lambda i, h: (0, h))],
        out_specs=pl.BlockSpec((1024, HALF), lambda i, h: (h * 10 + i, 0)),
        out_shape=jax.ShapeDtypeStruct((2 * NPAD, HALF), jnp.float32),
    )
    x1_st = tc1(agg1_st, eap, W_rel1, W_root1, b_rel1.reshape(1, HID))

    tc_layer = pl.pallas_call(
        _tc_layer_body,
        grid=(10, 2),
        in_specs=[pl.BlockSpec((1024, HALF), lambda i, h: (i, 0)),
                  pl.BlockSpec((1024, HALF), lambda i, h: (10 + i, 0)),
                  pl.BlockSpec((1024, HALF), lambda i, h: (i, 0)),
                  pl.BlockSpec((1024, HALF), lambda i, h: (10 + i, 0)),
                  pl.BlockSpec((HID, HALF), lambda i, h: (0, h)),
                  pl.BlockSpec((HID, HALF), lambda i, h: (0, h)),
                  pl.BlockSpec((1, HALF), lambda i, h: (0, h))],
        out_specs=pl.BlockSpec((1024, HALF), lambda i, h: (h * 10 + i, 0)),
        out_shape=jax.ShapeDtypeStruct((2 * NPAD, HALF), jnp.float32),
    )

    agg2_st = sc_seg(x1_st, srcp, dstp, zeros_rows)
    x2_st = tc_layer(agg2_st, agg2_st, x1_st, x1_st,
                     W_rel2, W_root2, b_rel2.reshape(1, HID))
    agg3_st = sc_seg(x2_st, srcp, dstp, zeros_rows)
    x3_st = tc_layer(agg3_st, agg3_st, x2_st, x2_st,
                     W_rel3, W_root3, b_rel3.reshape(1, HID))

    sc_pool = pl.kernel(
        _sc_pool_body,
        out_type=jax.ShapeDtypeStruct((384, HALF), jnp.float32),
        mesh=mesh,
        scratch_types=[pltpu.VMEM_SHARED((GPAD, HALF), jnp.float32),
                       pltpu.VMEM_SHARED((GPAD, HALF), jnp.float32),
                       pltpu.VMEM_SHARED((GPAD, HALF), jnp.float32),
                       pltpu.VMEM((NPAD,), jnp.int32),
                       pltpu.VMEM((EBP,), jnp.int32),
                       pltpu.VMEM((EBP,), jnp.int32),
                       pltpu.VMEM((EBP,), jnp.int32),
                       pltpu.VMEM((ROWS,), jnp.int32),
                       pltpu.VMEM((ROWS, HALF), jnp.float32)],
        compiler_params=sc_params,
    )
    pool = sc_pool(x3_st, srcp, dstp, batchpad, ones_rows, zeros_g)

    tc_head = pl.pallas_call(
        _tc_head_body,
        in_specs=[_full((384, HALF)),
                  _full((HID, HID)), _full((HID, HID)), _full((1, HID)),
                  _full((HID, HID)), _full((1, HID)),
                  _full((HID, HID)), _full((1, HID)),
                  _full((HID, HID)), _full((1, HID)),
                  _full((HID, HID)), _full((1, HID)),
                  _full((HID, 2)), _full((1, 2))],
        out_specs=pl.BlockSpec((G, 2), lambda: (0, 0)),
        out_shape=jax.ShapeDtypeStruct((G, 2), jnp.float32),
    )
    out = tc_head(pool, W_rel4, W_root4, b_rel4.reshape(1, HID),
                  W5, b5.reshape(1, HID), W6, b6.reshape(1, HID),
                  W7, b7.reshape(1, HID), W8, b8.reshape(1, HID),
                  W_out, b_out.reshape(1, 2))
    return out


# speedup vs baseline: 3.9302x; 1.5420x over previous
import dataclasses

import jax
import jax.numpy as jnp
from jax import lax
from jax.experimental import pallas as pl
from jax.experimental.pallas import tpu as pltpu
from jax.experimental.pallas import tpu_sc as plsc

N = 10000
E = 160000
HID = 256
HALF = 128
G = 64
NPAD = 10240
ROWS = NPAD // 16
EPT = NPAD
EPAD = 16 * EPT
DTRASH = 10200
EBS = 128
NBS = EPT // EBS
EBP = 256
NBP = EPT // EBP
GPAD = 72


def _sc_agg1_body(ea_hbm, srcp_hbm, dstp_hbm, agg1_out,
                  ea_v, acc_v, src_v, dst_v, out_v, stripe_v, red):
    c = lax.axis_index("c")
    s = lax.axis_index("s")
    pltpu.sync_copy(ea_hbm, ea_v)

    @pl.loop(0, NPAD // 16)
    def _(j):
        acc_v[pl.ds(j * 16, 16)] = jnp.zeros((16,), jnp.float32)

    pltpu.sync_copy(srcp_hbm.at[pl.ds(s * EPT, EPT)], src_v)
    pltpu.sync_copy(dstp_hbm.at[pl.ds(s * EPT, EPT)], dst_v)

    @pl.loop(0, EPT // 16)
    def _(j):
        s16 = src_v[pl.ds(j * 16, 16)]
        d16 = dst_v[pl.ds(j * 16, 16)]
        vals = plsc.load_gather(ea_v, [s16])
        plsc.addupdate_scatter(acc_v, [d16], vals)

    pltpu.sync_copy(acc_v, red.at[s])
    plsc.subcore_barrier()
    for r in range(16):
        pltpu.sync_copy(red.at[r, pl.ds(s * ROWS, ROWS)], stripe_v.at[r])

    @pl.loop(0, ROWS // 16)
    def _(j):
        tot = stripe_v[0, pl.ds(j * 16, 16)]
        for r in range(1, 16):
            tot = tot + stripe_v[r, pl.ds(j * 16, 16)]
        out_v[pl.ds(j * 16, 16)] = tot

    pltpu.sync_copy(out_v, agg1_out.at[pl.ds(c * NPAD + s * ROWS, ROWS)])


def _sc_seg_body(x_hbm, srcp_hbm, dstp_hbm, zeros_hbm, agg_out,
                 acc, isrc0, idst0, isrc1, idst1, rows0, rows1, sem0, sem1):
    c = lax.axis_index("c")
    s = lax.axis_index("s")
    rowoff = c * NPAD
    pltpu.sync_copy(zeros_hbm, acc.at[pl.ds(s * ROWS, ROWS)])
    plsc.subcore_barrier()

    def load_idx(isrc, idst, i):
        base = s * EPT + i * EBS
        pltpu.sync_copy(srcp_hbm.at[pl.ds(base, EBS)], isrc)
        pltpu.sync_copy(dstp_hbm.at[pl.ds(base, EBS)], idst)

        @pl.loop(0, EBS // 16)
        def _(j):
            isrc[pl.ds(j * 16, 16)] = isrc[pl.ds(j * 16, 16)] + rowoff

    load_idx(isrc0, idst0, 0)
    pltpu.async_copy(x_hbm.at[isrc0], rows0, sem0)

    @pl.loop(0, NBS // 2)
    def _(k):
        load_idx(isrc1, idst1, 2 * k + 1)
        pltpu.async_copy(x_hbm.at[isrc1], rows1, sem1)
        pltpu.make_async_copy(x_hbm.at[isrc0], rows0, sem0).wait()
        pltpu.sync_copy(rows0, acc.at[idst0], add=True)

        @pl.when(k < NBS // 2 - 1)
        def _():
            load_idx(isrc0, idst0, 2 * k + 2)
            pltpu.async_copy(x_hbm.at[isrc0], rows0, sem0)

        pltpu.make_async_copy(x_hbm.at[isrc1], rows1, sem1).wait()
        pltpu.sync_copy(rows1, acc.at[idst1], add=True)

    plsc.subcore_barrier()
    pltpu.sync_copy(acc.at[pl.ds(s * ROWS, ROWS)],
                    agg_out.at[pl.ds(rowoff + s * ROWS, ROWS)])


def _sc_pool_body(x_hbm, srcp_hbm, dstp_hbm, batchpad_hbm, ones_hbm, zeros_hbm,
                  pool_out,
                  accP, accX, accC, bpad_v, isrc0, idst0, isrc1, idst1, bg0, bg1,
                  bgn0, bgn1, bgn2, bgn3, bgn4,
                  rows0, rows1, sem0, sem1):
    c = lax.axis_index("c")
    s = lax.axis_index("s")
    rowoff = c * NPAD

    @pl.when(s == 0)
    def _():
        pltpu.sync_copy(zeros_hbm, accP)

    @pl.when(s == 1)
    def _():
        pltpu.sync_copy(zeros_hbm, accX)

    @pl.when(s == 2)
    def _():
        pltpu.sync_copy(zeros_hbm, accC)

    pltpu.sync_copy(batchpad_hbm, bpad_v)
    plsc.subcore_barrier()

    def prep(isrc, idst, bg, i):
        base = s * EPT + i * EBP
        pltpu.sync_copy(srcp_hbm.at[pl.ds(base, EBP)], isrc)
        pltpu.sync_copy(dstp_hbm.at[pl.ds(base, EBP)], idst)

        @pl.loop(0, EBP // 16)
        def _(j):
            d16 = idst[pl.ds(j * 16, 16)]
            bg[pl.ds(j * 16, 16)] = plsc.load_gather(bpad_v, [d16])
            isrc[pl.ds(j * 16, 16)] = isrc[pl.ds(j * 16, 16)] + rowoff

    prep(isrc0, idst0, bg0, 0)
    pltpu.async_copy(x_hbm.at[isrc0], rows0, sem0)

    @pl.loop(0, NBP // 2)
    def _(k):
        prep(isrc1, idst1, bg1, 2 * k + 1)
        pltpu.async_copy(x_hbm.at[isrc1], rows1, sem1)
        pltpu.make_async_copy(x_hbm.at[isrc0], rows0, sem0).wait()
        pltpu.sync_copy(rows0, accP.at[bg0], add=True)

        @pl.when(k < NBP // 2 - 1)
        def _():
            prep(isrc0, idst0, bg0, 2 * k + 2)
            pltpu.async_copy(x_hbm.at[isrc0], rows0, sem0)

        pltpu.make_async_copy(x_hbm.at[isrc1], rows1, sem1).wait()
        pltpu.sync_copy(rows1, accP.at[bg1], add=True)

    bgns = [bgn0, bgn1, bgn2, bgn3, bgn4]
    for t in range(5):
        @pl.loop(0, 8)
        def _(j, t=t):
            bgns[t][pl.ds(j * 16, 16)] = bpad_v[pl.ds(s * ROWS + t * 128 + j * 16, 16)]

    for t in range(5):
        pltpu.sync_copy(x_hbm.at[pl.ds(rowoff + s * ROWS + t * 128, 128)],
                        rows0.at[pl.ds(0, 128)])
        pltpu.sync_copy(rows0.at[pl.ds(0, 128)], accX.at[bgns[t]], add=True)

    @pl.when(c == 0)
    def _():
        pltpu.sync_copy(ones_hbm, rows0.at[pl.ds(0, 128)])
        for t in range(5):
            pltpu.sync_copy(rows0.at[pl.ds(0, 128)], accC.at[bgns[t]], add=True)

    plsc.subcore_barrier()

    @pl.when(s == 0)
    def _():
        pltpu.sync_copy(accP.at[pl.ds(0, G)], pool_out.at[pl.ds(c * 192, G)])

    @pl.when(s == 1)
    def _():
        pltpu.sync_copy(accX.at[pl.ds(0, G)], pool_out.at[pl.ds(c * 192 + G, G)])

    @pl.when(s == 2)
    def _():
        pltpu.sync_copy(accC.at[pl.ds(0, G)], pool_out.at[pl.ds(c * 192 + 2 * G, G)])


def _tc1_body(agg1_ref, ea_ref, wrel_ref, wroot_ref, b_ref, out_ref):
    a = agg1_ref[...]
    e = ea_ref[...]
    out_ref[...] = jnp.maximum(a * wrel_ref[...] + e * wroot_ref[...] + b_ref[...], 0.0)


def _tc_layer_body(agglo, agghi, xlo, xhi, wrel, wroot, b, out_ref):
    agg = jnp.concatenate([agglo[...], agghi[...]], axis=1)
    x = jnp.concatenate([xlo[...], xhi[...]], axis=1)
    res = (jnp.dot(agg, wrel[...], preferred_element_type=jnp.float32,
                   precision=lax.Precision.HIGHEST)
           + jnp.dot(x, wroot[...], preferred_element_type=jnp.float32,
                     precision=lax.Precision.HIGHEST)
           + b[...])
    out_ref[...] = jnp.maximum(res, 0.0)


def _tc_head_body(pool_ref, wrel4, wroot4, brel4,
                  w5, b5, w6, b6, w7, b7, w8, b8, wout, bout, out_ref):
    pool = pool_ref[...]
    P = jnp.concatenate([pool[0:G], pool[192:192 + G]], axis=1)
    X = jnp.concatenate([pool[G:2 * G], pool[192 + G:192 + 2 * G]], axis=1)
    cnt = pool[2 * G:3 * G, :1]

    def dot(a, b):
        return jnp.dot(a, b, preferred_element_type=jnp.float32,
                       precision=lax.Precision.HIGHEST)

    sums = dot(P, wrel4[...]) + cnt * brel4[...] + dot(X, wroot4[...])
    g = sums / jnp.maximum(cnt, 1.0)
    g = dot(g, w5[...]) + b5[...]
    g = dot(g, w6[...]) + b6[...]
    g = dot(g, w7[...]) + b7[...]
    g = dot(g, w8[...]) + b8[...]
    out_ref[...] = jax.nn.sigmoid(dot(g, wout[...]) + bout[...])


def _full(shape):
    return pl.BlockSpec(shape, lambda *_: tuple(0 for _ in shape))


def kernel(edge_attr, edge_index, batch,
           W_rel1, b_rel1, W_root1,
           W_rel2, b_rel2, W_root2,
           W_rel3, b_rel3, W_root3,
           W_rel4, b_rel4, W_root4,
           W5, b5, W6, b6, W7, b7, W8, b8,
           W_out, b_out):
    srcp = jnp.concatenate([edge_index[0], jnp.zeros((EPAD - E,), jnp.int32)])
    dstp = jnp.concatenate([edge_index[1], jnp.full((EPAD - E,), DTRASH, jnp.int32)])
    batchpad = jnp.concatenate([batch, jnp.full((NPAD - N,), G, jnp.int32)])
    eaflat = jnp.pad(edge_attr[:, 0], (0, NPAD - N))
    eap = jnp.pad(edge_attr, ((0, NPAD - N), (0, 0)))
    ones_rows = jnp.ones((128, HALF), jnp.float32)
    zeros_rows = jnp.zeros((ROWS, HALF), jnp.float32)
    zeros_g = jnp.zeros((GPAD, HALF), jnp.float32)

    mesh = plsc.VectorSubcoreMesh(core_axis_name="c", subcore_axis_name="s")
    sc_params = pltpu.CompilerParams()
    if "needs_layout_passes" in pltpu.CompilerParams.__dataclass_fields__:
        sc_params = dataclasses.replace(sc_params, needs_layout_passes=False)

    sc_agg1 = pl.kernel(
        _sc_agg1_body,
        out_type=jax.ShapeDtypeStruct((2 * NPAD,), jnp.float32),
        mesh=mesh,
        scratch_types=[pltpu.VMEM((NPAD,), jnp.float32),
                       pltpu.VMEM((NPAD,), jnp.float32),
                       pltpu.VMEM((EPT,), jnp.int32),
                       pltpu.VMEM((EPT,), jnp.int32),
                       pltpu.VMEM((ROWS,), jnp.float32),
                       pltpu.VMEM((16, ROWS), jnp.float32),
                       pltpu.VMEM_SHARED((16, NPAD), jnp.float32)],
        compiler_params=sc_params,
    )
    agg1 = sc_agg1(eaflat, srcp, dstp)[:NPAD].reshape(NPAD, 1)

    tc1 = pl.pallas_call(
        _tc1_body,
        grid=(10, 2),
        in_specs=[pl.BlockSpec((1024, 1), lambda i, h: (i, 0)),
                  pl.BlockSpec((1024, 1), lambda i, h: (i, 0)),
                  pl.BlockSpec((1, HALF), lambda i, h: (0, h)),
                  pl.BlockSpec((1, HALF), lambda i, h: (0, h)),
                  pl.BlockSpec((1, HALF), lambda i, h: (0, h))],
        out_specs=pl.BlockSpec((1024, HALF), lambda i, h: (h * 10 + i, 0)),
        out_shape=jax.ShapeDtypeStruct((2 * NPAD, HALF), jnp.float32),
    )
    x1_st = tc1(agg1, eap, W_rel1, W_root1, b_rel1.reshape(1, HID))

    sc_seg = pl.kernel(
        _sc_seg_body,
        out_type=jax.ShapeDtypeStruct((2 * NPAD, HALF), jnp.float32),
        mesh=mesh,
        scratch_types=[pltpu.VMEM_SHARED((NPAD, HALF), jnp.float32),
                       pltpu.VMEM((EBS,), jnp.int32),
                       pltpu.VMEM((EBS,), jnp.int32),
                       pltpu.VMEM((EBS,), jnp.int32),
                       pltpu.VMEM((EBS,), jnp.int32),
                       pltpu.VMEM((EBS, HALF), jnp.float32),
                       pltpu.VMEM((EBS, HALF), jnp.float32),
                       pltpu.SemaphoreType.DMA,
                       pltpu.SemaphoreType.DMA],
        compiler_params=sc_params,
    )

    tc_layer = pl.pallas_call(
        _tc_layer_body,
        grid=(10, 2),
        in_specs=[pl.BlockSpec((1024, HALF), lambda i, h: (i, 0)),
                  pl.BlockSpec((1024, HALF), lambda i, h: (10 + i, 0)),
                  pl.BlockSpec((1024, HALF), lambda i, h: (i, 0)),
                  pl.BlockSpec((1024, HALF), lambda i, h: (10 + i, 0)),
                  pl.BlockSpec((HID, HALF), lambda i, h: (0, h)),
                  pl.BlockSpec((HID, HALF), lambda i, h: (0, h)),
                  pl.BlockSpec((1, HALF), lambda i, h: (0, h))],
        out_specs=pl.BlockSpec((1024, HALF), lambda i, h: (h * 10 + i, 0)),
        out_shape=jax.ShapeDtypeStruct((2 * NPAD, HALF), jnp.float32),
    )

    agg2_st = sc_seg(x1_st, srcp, dstp, zeros_rows)
    x2_st = tc_layer(agg2_st, agg2_st, x1_st, x1_st,
                     W_rel2, W_root2, b_rel2.reshape(1, HID))
    agg3_st = sc_seg(x2_st, srcp, dstp, zeros_rows)
    x3_st = tc_layer(agg3_st, agg3_st, x2_st, x2_st,
                     W_rel3, W_root3, b_rel3.reshape(1, HID))

    sc_pool = pl.kernel(
        _sc_pool_body,
        out_type=jax.ShapeDtypeStruct((384, HALF), jnp.float32),
        mesh=mesh,
        scratch_types=[pltpu.VMEM_SHARED((GPAD, HALF), jnp.float32),
                       pltpu.VMEM_SHARED((GPAD, HALF), jnp.float32),
                       pltpu.VMEM_SHARED((GPAD, HALF), jnp.float32),
                       pltpu.VMEM((NPAD,), jnp.int32),
                       pltpu.VMEM((EBP,), jnp.int32),
                       pltpu.VMEM((EBP,), jnp.int32),
                       pltpu.VMEM((EBP,), jnp.int32),
                       pltpu.VMEM((EBP,), jnp.int32),
                       pltpu.VMEM((EBP,), jnp.int32),
                       pltpu.VMEM((EBP,), jnp.int32),
                       pltpu.VMEM((128,), jnp.int32),
                       pltpu.VMEM((128,), jnp.int32),
                       pltpu.VMEM((128,), jnp.int32),
                       pltpu.VMEM((128,), jnp.int32),
                       pltpu.VMEM((128,), jnp.int32),
                       pltpu.VMEM((EBP, HALF), jnp.float32),
                       pltpu.VMEM((EBP, HALF), jnp.float32),
                       pltpu.SemaphoreType.DMA,
                       pltpu.SemaphoreType.DMA],
        compiler_params=sc_params,
    )
    pool = sc_pool(x3_st, srcp, dstp, batchpad, ones_rows, zeros_g)

    tc_head = pl.pallas_call(
        _tc_head_body,
        in_specs=[_full((384, HALF)),
                  _full((HID, HID)), _full((HID, HID)), _full((1, HID)),
                  _full((HID, HID)), _full((1, HID)),
                  _full((HID, HID)), _full((1, HID)),
                  _full((HID, HID)), _full((1, HID)),
                  _full((HID, HID)), _full((1, HID)),
                  _full((HID, 2)), _full((1, 2))],
        out_specs=pl.BlockSpec((G, 2), lambda: (0, 0)),
        out_shape=jax.ShapeDtypeStruct((G, 2), jnp.float32),
    )
    out = tc_head(pool, W_rel4, W_root4, b_rel4.reshape(1, HID),
                  W5, b5.reshape(1, HID), W6, b6.reshape(1, HID),
                  W7, b7.reshape(1, HID), W8, b8.reshape(1, HID),
                  W_out, b_out.reshape(1, 2))
    return out
